# baseline (speedup 1.0000x reference)
import functools

import jax
import jax.numpy as jnp
import numpy as np
from jax import lax
from jax.experimental import pallas as pl
from jax.experimental.pallas import tpu as pltpu
from jax.experimental.pallas import tpu_sc as plsc

N_AG = 128
N_TASK = 2048
N_NODES = N_AG + N_TASK
N_EDGES = 262144
D = 512

SC_CORES = 2
SC_SUBCORES = 16
LANES = 16

EDGES_PER_TILE = N_EDGES // SC_SUBCORES
IDX_ROWS = EDGES_PER_TILE // 128
CHUNK_ROWS = N_NODES // 4
CHUNK = CHUNK_ROWS * N_NODES


def _threefry2x32(k1, k2, x0, x1):
    rot = ((13, 15, 26, 6), (17, 29, 16, 24))
    ks = (np.uint32(k1), np.uint32(k2),
          np.uint32(np.uint32(k1) ^ np.uint32(k2) ^ np.uint32(0x1BD11BDA)))
    x0 = (x0 + ks[0]).astype(np.uint32)
    x1 = (x1 + ks[1]).astype(np.uint32)
    for g in range(5):
        for r in rot[g % 2]:
            x0 = (x0 + x1).astype(np.uint32)
            x1 = ((x1 << np.uint32(r)) | (x1 >> np.uint32(32 - r))).astype(
                np.uint32)
            x1 = (x0 ^ x1).astype(np.uint32)
        x0 = (x0 + ks[(g + 1) % 3]).astype(np.uint32)
        x1 = (x1 + ks[(g + 2) % 3] + np.uint32(g + 1)).astype(np.uint32)
    return x0, x1


def _make_gumbel_table() -> np.ndarray:
    tiny = np.float32(np.finfo(np.float32).tiny)
    k1, k2 = np.uint32(0), np.uint32(42)
    rows = []
    for _ in range(N_AG):
        b1, b2 = _threefry2x32(k1, k2, np.zeros(2, np.uint32),
                               np.arange(2, dtype=np.uint32))
        k1, k2 = b1[0], b2[0]
        s1, s2 = b1[1], b2[1]
        r1, r2 = _threefry2x32(s1, s2, np.zeros(N_TASK, np.uint32),
                               np.arange(N_TASK, dtype=np.uint32))
        bits = (r1 ^ r2).astype(np.uint32)
        fb = ((bits >> np.uint32(9)) | np.uint32(0x3F800000)).astype(np.uint32)
        u = fb.view(np.float32) - np.float32(1.0)
        u = np.maximum(tiny, (u * (np.float32(1.0) - tiny) + tiny))
        g = -np.log(-np.log(u.astype(np.float64)))
        rows.append(g.astype(np.float32))
    return np.stack(rows)


_GUMBEL = _make_gumbel_table()


def _build_counts(edge_src, edge_dst, zeros):
    mesh = plsc.VectorSubcoreMesh(core_axis_name="c", subcore_axis_name="s")

    @functools.partial(
        pl.kernel,
        mesh=mesh,
        out_type=jax.ShapeDtypeStruct((N_NODES * N_NODES,), jnp.float32),
        scratch_types=[
            pltpu.VMEM((EDGES_PER_TILE,), jnp.int32),
            pltpu.VMEM((EDGES_PER_TILE,), jnp.int32),
            pltpu.VMEM((IDX_ROWS, 128), jnp.int32),
            pltpu.VMEM((8, 128), jnp.float32),
            pltpu.VMEM((128,), jnp.float32),
            pltpu.VMEM_SHARED((CHUNK + 8,), jnp.float32),
            pltpu.SemaphoreType.DMA,
        ],
    )
    def counts_kernel(src_hbm, dst_hbm, z_hbm, a_hbm, src_v, dst_v, flat_v,
                      ones_v, drain_v, acc, sem):
        c = lax.axis_index("c")
        s = lax.axis_index("s")
        pltpu.sync_copy(src_hbm.at[s], src_v)
        pltpu.sync_copy(dst_hbm.at[s], dst_v)

        for r in range(8):
            for q in range(128 // LANES):
                ones_v[r, pl.ds(q * LANES, LANES)] = jnp.full(
                    (LANES,), 1.0, jnp.float32)
        for k in range(2):
            chunk_id = c * 2 + k
            lo = chunk_id * CHUNK_ROWS

            @pl.when(s == 0)
            def _():
                pltpu.sync_copy(z_hbm, acc)

            plsc.subcore_barrier()

            def row_body(r, _):
                for q in range(128 // LANES):
                    off = r * 128 + q * LANES
                    sv = src_v[pl.ds(off, LANES)]
                    dv = dst_v[pl.ds(off, LANES)]
                    rel = dv - lo
                    inr = (rel >= 0) & (rel < CHUNK_ROWS)
                    flat = jnp.where(inr, rel * N_NODES + sv, CHUNK)
                    flat_v[r, pl.ds(q * LANES, LANES)] = flat
                return 0

            lax.fori_loop(0, IDX_ROWS, row_body, 0)

            def sc_body(j, _):
                descs = [
                    pltpu.async_copy(ones_v.at[t],
                                     acc.at[flat_v.at[j * 8 + t]], sem,
                                     add=True)
                    for t in range(8)
                ]
                for dsc in descs:
                    dsc.wait()
                return 0

            lax.fori_loop(0, IDX_ROWS // 8, sc_body, 0)
            for j in range(IDX_ROWS - 4, IDX_ROWS):
                pltpu.sync_copy(acc.at[flat_v.at[j]], drain_v)
            plsc.subcore_barrier()

            @pl.when(s == 0)
            def _():
                pl.delay(2000)
                pltpu.sync_copy(acc.at[pl.ds(0, CHUNK)],
                                a_hbm.at[pl.ds(chunk_id * CHUNK, CHUNK)])

            plsc.subcore_barrier()

    return counts_kernel(edge_src, edge_dst, zeros)


def _embed(loc, W_embed, b_embed2d, interpret=False):
    def body(loc_ref, we_ref, be_ref, h_ref):
        x0 = loc_ref[:, 0:1].astype(jnp.bfloat16).astype(jnp.float32)
        x1 = loc_ref[:, 1:2].astype(jnp.bfloat16).astype(jnp.float32)
        w0 = we_ref[0:1, :].astype(jnp.bfloat16).astype(jnp.float32)
        w1 = we_ref[1:2, :].astype(jnp.bfloat16).astype(jnp.float32)
        h_ref[...] = (x0 * w0 + x1 * w1) + be_ref[...]

    return pl.pallas_call(
        body,
        out_shape=jax.ShapeDtypeStruct((N_NODES, D), jnp.float32),
        interpret=interpret,
    )(loc, W_embed, b_embed2d)


def _gnn(A, h, W_self, W_nei, b_gnn2d, interpret=False):
    nblk = N_NODES // 128

    def body(a_ref, h_ref, ws_ref, wn_ref, bg_ref, h2_ref):
        i = pl.program_id(0)
        a_blk = a_ref[...]
        deg = jnp.sum(a_blk, axis=1, keepdims=True)
        agg = jax.lax.dot_general(
            a_blk, h_ref[...], (((1,), (0,)), ((), ())),
            precision=jax.lax.Precision.HIGHEST,
            preferred_element_type=jnp.float32)
        norm = agg / jnp.maximum(deg, 1.0)
        h_blk = h_ref[pl.ds(i * 128, 128), :]
        pre = (jax.lax.dot_general(
            h_blk, ws_ref[...], (((1,), (0,)), ((), ())),
            preferred_element_type=jnp.float32) +
               jax.lax.dot_general(
            norm, wn_ref[...], (((1,), (0,)), ((), ())),
            preferred_element_type=jnp.float32) + bg_ref[...])
        h2_ref[...] = jnp.maximum(pre, 0.0) + h_blk

    return pl.pallas_call(
        body,
        grid=(nblk,),
        in_specs=[
            pl.BlockSpec((128, N_NODES), lambda i: (i, 0)),
            pl.BlockSpec((N_NODES, D), lambda i: (0, 0)),
            pl.BlockSpec((D, D), lambda i: (0, 0)),
            pl.BlockSpec((D, D), lambda i: (0, 0)),
            pl.BlockSpec((1, D), lambda i: (0, 0)),
        ],
        out_specs=pl.BlockSpec((128, D), lambda i: (i, 0)),
        out_shape=jax.ShapeDtypeStruct((N_NODES, D), jnp.float32),
        interpret=interpret,
    )(A, h, W_self, W_nei, b_gnn2d)


def _sample(h2, W_bi, gum, ag_order, continuing, prev, interpret=False):
    def body(h2_ref, wb_ref, gum_ref, ago_ref, cont_ref, prev_ref, out_ref,
             lut_ref):
        ag = h2_ref[0:N_AG, :]
        tasks = h2_ref[N_AG:N_NODES, :]
        t = jax.lax.dot_general(
            ag, wb_ref[...], (((1,), (0,)), ((), ())),
            preferred_element_type=jnp.float32)
        logits = jax.lax.dot_general(
            t, tasks, (((1,), (1,)), ((), ())),
            preferred_element_type=jnp.float32)
        m = jnp.max(logits, axis=1, keepdims=True)
        e = jnp.exp(logits - m)
        pol = e / jnp.sum(e, axis=1, keepdims=True)
        colid = lax.broadcasted_iota(jnp.int32, (N_AG, N_TASK), 1)
        pol = jnp.where(colid == N_TASK - 1, jnp.float32(1e-5), pol)
        lut_ref[...] = jnp.log(pol + 1e-12)
        log_masked = jnp.log(jnp.float32(1e-12))

        col1 = lax.broadcasted_iota(jnp.int32, (1, N_TASK), 1)
        outid = lax.broadcasted_iota(jnp.int32, (1, N_AG), 1)

        def step(itr, carry):
            mask, acts = carry
            a = ago_ref[itr]
            base = lut_ref[pl.ds(a, 1), :]
            g = gum_ref[pl.ds(itr, 1), :]
            scores = jnp.where(mask != 0, log_masked, base) + g
            mx = jnp.max(scores)
            action = jnp.min(
                jnp.where(scores == mx, col1, N_TASK)).astype(jnp.int32)
            action = jnp.where(cont_ref[a] != 0, prev_ref[a], action)
            mask = mask | ((col1 == action) &
                           (action != N_TASK - 1)).astype(jnp.int32)
            acts = jnp.where(outid == itr, action, acts)
            return mask, acts

        mask0 = jnp.zeros((1, N_TASK), jnp.int32)
        acts0 = jnp.zeros((1, N_AG), jnp.int32)
        _, acts = lax.fori_loop(0, N_AG, step, (mask0, acts0))
        out_ref[...] = acts

    return pl.pallas_call(
        body,
        in_specs=[
            pl.BlockSpec(memory_space=pltpu.VMEM),
            pl.BlockSpec(memory_space=pltpu.VMEM),
            pl.BlockSpec(memory_space=pltpu.VMEM),
            pl.BlockSpec(memory_space=pltpu.SMEM),
            pl.BlockSpec(memory_space=pltpu.SMEM),
            pl.BlockSpec(memory_space=pltpu.SMEM),
        ],
        out_specs=pl.BlockSpec(memory_space=pltpu.VMEM),
        out_shape=jax.ShapeDtypeStruct((1, N_AG), jnp.int32),
        scratch_shapes=[pltpu.VMEM((N_AG, N_TASK), jnp.float32)],
        interpret=interpret,
    )(h2, W_bi, gum, ag_order, continuing, prev)


def kernel(loc, W_embed, b_embed, W_self, W_nei, b_gnn, W_bi, edge_index,
           ag_order, continuing_ag, joint_action_prev):
    edge_src = edge_index[0].reshape(SC_SUBCORES, EDGES_PER_TILE)
    edge_dst = edge_index[1].reshape(SC_SUBCORES, EDGES_PER_TILE)
    zeros = jnp.zeros((CHUNK + 8,), jnp.float32)
    a_flat = _build_counts(edge_src.astype(jnp.int32),
                           edge_dst.astype(jnp.int32), zeros)
    A = a_flat.reshape(N_NODES, N_NODES)
    h = _embed(loc, W_embed, b_embed.reshape(1, D))
    h2 = _gnn(A, h, W_self, W_nei, b_gnn.reshape(1, D))
    acts = _sample(h2, W_bi, jnp.asarray(_GUMBEL),
                   ag_order.astype(jnp.int32),
                   continuing_ag.astype(jnp.int32),
                   joint_action_prev.astype(jnp.int32))
    return acts.reshape(N_AG)


# speedup vs baseline: 3.1446x; 3.1446x over previous
import functools

import jax
import jax.numpy as jnp
import numpy as np
from jax import lax
from jax.experimental import pallas as pl
from jax.experimental.pallas import tpu as pltpu
from jax.experimental.pallas import tpu_sc as plsc

N_AG = 128
N_TASK = 2048
N_NODES = N_AG + N_TASK
N_EDGES = 262144
D = 512

SC_CORES = 2
SC_SUBCORES = 16
LANES = 16

EDGES_PER_TILE = N_EDGES // SC_SUBCORES
IDX_ROWS = EDGES_PER_TILE // 128
CHUNK_ROWS = N_NODES // 4
CHUNK = CHUNK_ROWS * N_NODES


def _threefry2x32(k1, k2, x0, x1):
    rot = ((13, 15, 26, 6), (17, 29, 16, 24))
    ks = (np.uint32(k1), np.uint32(k2),
          np.uint32(np.uint32(k1) ^ np.uint32(k2) ^ np.uint32(0x1BD11BDA)))
    x0 = (x0 + ks[0]).astype(np.uint32)
    x1 = (x1 + ks[1]).astype(np.uint32)
    for g in range(5):
        for r in rot[g % 2]:
            x0 = (x0 + x1).astype(np.uint32)
            x1 = ((x1 << np.uint32(r)) | (x1 >> np.uint32(32 - r))).astype(
                np.uint32)
            x1 = (x0 ^ x1).astype(np.uint32)
        x0 = (x0 + ks[(g + 1) % 3]).astype(np.uint32)
        x1 = (x1 + ks[(g + 2) % 3] + np.uint32(g + 1)).astype(np.uint32)
    return x0, x1


def _make_gumbel_table() -> np.ndarray:
    tiny = np.float32(np.finfo(np.float32).tiny)
    k1, k2 = np.uint32(0), np.uint32(42)
    rows = []
    for _ in range(N_AG):
        b1, b2 = _threefry2x32(k1, k2, np.zeros(2, np.uint32),
                               np.arange(2, dtype=np.uint32))
        k1, k2 = b1[0], b2[0]
        s1, s2 = b1[1], b2[1]
        r1, r2 = _threefry2x32(s1, s2, np.zeros(N_TASK, np.uint32),
                               np.arange(N_TASK, dtype=np.uint32))
        bits = (r1 ^ r2).astype(np.uint32)
        fb = ((bits >> np.uint32(9)) | np.uint32(0x3F800000)).astype(np.uint32)
        u = fb.view(np.float32) - np.float32(1.0)
        u = np.maximum(tiny, (u * (np.float32(1.0) - tiny) + tiny))
        g = -np.log(-np.log(u.astype(np.float64)))
        rows.append(g.astype(np.float32))
    return np.stack(rows)


_GUMBEL = _make_gumbel_table()


def _build_counts(edge_src, edge_dst, zeros):
    mesh = plsc.VectorSubcoreMesh(core_axis_name="c", subcore_axis_name="s")

    @functools.partial(
        pl.kernel,
        mesh=mesh,
        out_type=jax.ShapeDtypeStruct((N_NODES * N_NODES,), jnp.float32),
        scratch_types=[
            pltpu.VMEM((EDGES_PER_TILE,), jnp.int32),
            pltpu.VMEM((EDGES_PER_TILE,), jnp.int32),
            pltpu.VMEM((IDX_ROWS, 128), jnp.int32),
            pltpu.VMEM((8, 128), jnp.float32),
            pltpu.VMEM((128,), jnp.float32),
            pltpu.VMEM_SHARED((CHUNK + SC_SUBCORES * 128,), jnp.float32),
            pltpu.SemaphoreType.DMA,
        ],
    )
    def counts_kernel(src_hbm, dst_hbm, z_hbm, a_hbm, src_v, dst_v, flat_v,
                      ones_v, drain_v, acc, sem):
        c = lax.axis_index("c")
        s = lax.axis_index("s")
        pltpu.sync_copy(src_hbm.at[s], src_v)
        pltpu.sync_copy(dst_hbm.at[s], dst_v)

        for r in range(8):
            for q in range(128 // LANES):
                ones_v[r, pl.ds(q * LANES, LANES)] = jnp.full(
                    (LANES,), 1.0, jnp.float32)
        for k in range(2):
            chunk_id = c * 2 + k
            lo = chunk_id * CHUNK_ROWS

            @pl.when(s == 0)
            def _():
                pltpu.sync_copy(z_hbm, acc)

            plsc.subcore_barrier()

            lane = lax.iota(jnp.int32, LANES)

            def row_body(r, _):
                for q in range(128 // LANES):
                    off = r * 128 + q * LANES
                    sv = src_v[pl.ds(off, LANES)]
                    dv = dst_v[pl.ds(off, LANES)]
                    rel = dv - lo
                    inr = (rel >= 0) & (rel < CHUNK_ROWS)
                    trash = CHUNK + s * 128 + q * LANES + lane
                    flat = jnp.where(inr, rel * N_NODES + sv, trash)
                    flat_v[r, pl.ds(q * LANES, LANES)] = flat
                return 0

            lax.fori_loop(0, IDX_ROWS, row_body, 0)

            def sc_body(j, _):
                descs = [
                    pltpu.async_copy(ones_v.at[t],
                                     acc.at[flat_v.at[j * 8 + t]], sem,
                                     add=True)
                    for t in range(8)
                ]
                for dsc in descs:
                    dsc.wait()
                return 0

            lax.fori_loop(0, IDX_ROWS // 8, sc_body, 0)
            for j in range(IDX_ROWS - 4, IDX_ROWS):
                pltpu.sync_copy(acc.at[flat_v.at[j]], drain_v)
            plsc.subcore_barrier()

            @pl.when(s == 0)
            def _():
                pl.delay(2000)
                pltpu.sync_copy(acc.at[pl.ds(0, CHUNK)],
                                a_hbm.at[pl.ds(chunk_id * CHUNK, CHUNK)])

            plsc.subcore_barrier()

    return counts_kernel(edge_src, edge_dst, zeros)


def _embed(loc, W_embed, b_embed2d, interpret=False):
    def body(loc_ref, we_ref, be_ref, h_ref):
        x0 = loc_ref[:, 0:1].astype(jnp.bfloat16).astype(jnp.float32)
        x1 = loc_ref[:, 1:2].astype(jnp.bfloat16).astype(jnp.float32)
        w0 = we_ref[0:1, :].astype(jnp.bfloat16).astype(jnp.float32)
        w1 = we_ref[1:2, :].astype(jnp.bfloat16).astype(jnp.float32)
        h_ref[...] = (x0 * w0 + x1 * w1) + be_ref[...]

    return pl.pallas_call(
        body,
        out_shape=jax.ShapeDtypeStruct((N_NODES, D), jnp.float32),
        interpret=interpret,
    )(loc, W_embed, b_embed2d)


def _gnn(A, h, W_self, W_nei, b_gnn2d, interpret=False):
    nblk = N_NODES // 128

    def body(a_ref, h_ref, ws_ref, wn_ref, bg_ref, h2_ref):
        i = pl.program_id(0)
        a_blk = a_ref[...]
        deg = jnp.sum(a_blk, axis=1, keepdims=True)
        agg = jax.lax.dot_general(
            a_blk, h_ref[...], (((1,), (0,)), ((), ())),
            precision=jax.lax.Precision.HIGHEST,
            preferred_element_type=jnp.float32)
        norm = agg / jnp.maximum(deg, 1.0)
        h_blk = h_ref[pl.ds(i * 128, 128), :]
        pre = (jax.lax.dot_general(
            h_blk, ws_ref[...], (((1,), (0,)), ((), ())),
            preferred_element_type=jnp.float32) +
               jax.lax.dot_general(
            norm, wn_ref[...], (((1,), (0,)), ((), ())),
            preferred_element_type=jnp.float32) + bg_ref[...])
        h2_ref[...] = jnp.maximum(pre, 0.0) + h_blk

    return pl.pallas_call(
        body,
        grid=(nblk,),
        in_specs=[
            pl.BlockSpec((128, N_NODES), lambda i: (i, 0)),
            pl.BlockSpec((N_NODES, D), lambda i: (0, 0)),
            pl.BlockSpec((D, D), lambda i: (0, 0)),
            pl.BlockSpec((D, D), lambda i: (0, 0)),
            pl.BlockSpec((1, D), lambda i: (0, 0)),
        ],
        out_specs=pl.BlockSpec((128, D), lambda i: (i, 0)),
        out_shape=jax.ShapeDtypeStruct((N_NODES, D), jnp.float32),
        interpret=interpret,
    )(A, h, W_self, W_nei, b_gnn2d)


def _sample(h2, W_bi, gum, ag_order, continuing, prev, interpret=False):
    def body(h2_ref, wb_ref, gum_ref, ago_ref, cont_ref, prev_ref, out_ref,
             lut_ref):
        ag = h2_ref[0:N_AG, :]
        tasks = h2_ref[N_AG:N_NODES, :]
        t = jax.lax.dot_general(
            ag, wb_ref[...], (((1,), (0,)), ((), ())),
            preferred_element_type=jnp.float32)
        logits = jax.lax.dot_general(
            t, tasks, (((1,), (1,)), ((), ())),
            preferred_element_type=jnp.float32)
        m = jnp.max(logits, axis=1, keepdims=True)
        e = jnp.exp(logits - m)
        pol = e / jnp.sum(e, axis=1, keepdims=True)
        colid = lax.broadcasted_iota(jnp.int32, (N_AG, N_TASK), 1)
        pol = jnp.where(colid == N_TASK - 1, jnp.float32(1e-5), pol)
        lut_ref[...] = jnp.log(pol + 1e-12)
        log_masked = jnp.log(jnp.float32(1e-12))

        col1 = lax.broadcasted_iota(jnp.int32, (1, N_TASK), 1)
        outid = lax.broadcasted_iota(jnp.int32, (1, N_AG), 1)

        def step(itr, carry):
            mask, acts = carry
            a = ago_ref[itr]
            base = lut_ref[pl.ds(a, 1), :]
            g = gum_ref[pl.ds(itr, 1), :]
            scores = jnp.where(mask != 0, log_masked, base) + g
            mx = jnp.max(scores)
            action = jnp.min(
                jnp.where(scores == mx, col1, N_TASK)).astype(jnp.int32)
            action = jnp.where(cont_ref[a] != 0, prev_ref[a], action)
            mask = mask | ((col1 == action) &
                           (action != N_TASK - 1)).astype(jnp.int32)
            acts = jnp.where(outid == itr, action, acts)
            return mask, acts

        mask0 = jnp.zeros((1, N_TASK), jnp.int32)
        acts0 = jnp.zeros((1, N_AG), jnp.int32)
        _, acts = lax.fori_loop(0, N_AG, step, (mask0, acts0))
        out_ref[...] = acts

    return pl.pallas_call(
        body,
        in_specs=[
            pl.BlockSpec(memory_space=pltpu.VMEM),
            pl.BlockSpec(memory_space=pltpu.VMEM),
            pl.BlockSpec(memory_space=pltpu.VMEM),
            pl.BlockSpec(memory_space=pltpu.SMEM),
            pl.BlockSpec(memory_space=pltpu.SMEM),
            pl.BlockSpec(memory_space=pltpu.SMEM),
        ],
        out_specs=pl.BlockSpec(memory_space=pltpu.VMEM),
        out_shape=jax.ShapeDtypeStruct((1, N_AG), jnp.int32),
        scratch_shapes=[pltpu.VMEM((N_AG, N_TASK), jnp.float32)],
        interpret=interpret,
    )(h2, W_bi, gum, ag_order, continuing, prev)


def kernel(loc, W_embed, b_embed, W_self, W_nei, b_gnn, W_bi, edge_index,
           ag_order, continuing_ag, joint_action_prev):
    edge_src = edge_index[0].reshape(SC_SUBCORES, EDGES_PER_TILE)
    edge_dst = edge_index[1].reshape(SC_SUBCORES, EDGES_PER_TILE)
    zeros = jnp.zeros((CHUNK + SC_SUBCORES * 128,), jnp.float32)
    a_flat = _build_counts(edge_src.astype(jnp.int32),
                           edge_dst.astype(jnp.int32), zeros)
    A = a_flat.reshape(N_NODES, N_NODES)
    h = _embed(loc, W_embed, b_embed.reshape(1, D))
    h2 = _gnn(A, h, W_self, W_nei, b_gnn.reshape(1, D))
    acts = _sample(h2, W_bi, jnp.asarray(_GUMBEL),
                   ag_order.astype(jnp.int32),
                   continuing_ag.astype(jnp.int32),
                   joint_action_prev.astype(jnp.int32))
    return acts.reshape(N_AG)


# speedup vs baseline: 3.2073x; 1.0199x over previous
import functools

import jax
import jax.numpy as jnp
import numpy as np
from jax import lax
from jax.experimental import pallas as pl
from jax.experimental.pallas import tpu as pltpu
from jax.experimental.pallas import tpu_sc as plsc

N_AG = 128
N_TASK = 2048
N_NODES = N_AG + N_TASK
N_EDGES = 262144
D = 512

SC_CORES = 2
SC_SUBCORES = 16
LANES = 16

EDGES_PER_TILE = N_EDGES // SC_SUBCORES
IDX_ROWS = EDGES_PER_TILE // 128
CHUNK_ROWS = N_NODES // 4
CHUNK = CHUNK_ROWS * N_NODES


def _threefry2x32(k1, k2, x0, x1):
    rot = ((13, 15, 26, 6), (17, 29, 16, 24))
    ks = (np.uint32(k1), np.uint32(k2),
          np.uint32(np.uint32(k1) ^ np.uint32(k2) ^ np.uint32(0x1BD11BDA)))
    x0 = (x0 + ks[0]).astype(np.uint32)
    x1 = (x1 + ks[1]).astype(np.uint32)
    for g in range(5):
        for r in rot[g % 2]:
            x0 = (x0 + x1).astype(np.uint32)
            x1 = ((x1 << np.uint32(r)) | (x1 >> np.uint32(32 - r))).astype(
                np.uint32)
            x1 = (x0 ^ x1).astype(np.uint32)
        x0 = (x0 + ks[(g + 1) % 3]).astype(np.uint32)
        x1 = (x1 + ks[(g + 2) % 3] + np.uint32(g + 1)).astype(np.uint32)
    return x0, x1


def _make_gumbel_table() -> np.ndarray:
    tiny = np.float32(np.finfo(np.float32).tiny)
    k1, k2 = np.uint32(0), np.uint32(42)
    rows = []
    for _ in range(N_AG):
        b1, b2 = _threefry2x32(k1, k2, np.zeros(2, np.uint32),
                               np.arange(2, dtype=np.uint32))
        k1, k2 = b1[0], b2[0]
        s1, s2 = b1[1], b2[1]
        r1, r2 = _threefry2x32(s1, s2, np.zeros(N_TASK, np.uint32),
                               np.arange(N_TASK, dtype=np.uint32))
        bits = (r1 ^ r2).astype(np.uint32)
        fb = ((bits >> np.uint32(9)) | np.uint32(0x3F800000)).astype(np.uint32)
        u = fb.view(np.float32) - np.float32(1.0)
        u = np.maximum(tiny, (u * (np.float32(1.0) - tiny) + tiny))
        g = -np.log(-np.log(u.astype(np.float64)))
        rows.append(g.astype(np.float32))
    return np.stack(rows)


_GUMBEL = _make_gumbel_table()


ACC_LEN = CHUNK + SC_SUBCORES * 128
INIT_SLICE = ACC_LEN // SC_SUBCORES
OUT_SLICE = CHUNK // SC_SUBCORES


def _build_counts(edge_src, edge_dst, zeros):
    mesh = plsc.VectorSubcoreMesh(core_axis_name="c", subcore_axis_name="s")

    @functools.partial(
        pl.kernel,
        mesh=mesh,
        out_type=jax.ShapeDtypeStruct((N_NODES * N_NODES,), jnp.float32),
        scratch_types=[
            pltpu.VMEM((EDGES_PER_TILE,), jnp.int32),
            pltpu.VMEM((EDGES_PER_TILE,), jnp.int32),
            pltpu.VMEM((IDX_ROWS, 128), jnp.int32),
            pltpu.VMEM((8, 128), jnp.float32),
            pltpu.VMEM((128,), jnp.float32),
            pltpu.VMEM_SHARED((ACC_LEN,), jnp.float32),
            pltpu.SemaphoreType.DMA,
        ],
    )
    def counts_kernel(src_hbm, dst_hbm, z_hbm, a_hbm, src_v, dst_v, flat_v,
                      ones_v, drain_v, acc, sem):
        c = lax.axis_index("c")
        s = lax.axis_index("s")
        pltpu.sync_copy(src_hbm.at[s], src_v)
        pltpu.sync_copy(dst_hbm.at[s], dst_v)

        for r in range(8):
            for q in range(128 // LANES):
                ones_v[r, pl.ds(q * LANES, LANES)] = jnp.full(
                    (LANES,), 1.0, jnp.float32)

        for k in range(2):
            chunk_id = c * 2 + k
            lo = chunk_id * CHUNK_ROWS

            base = s * INIT_SLICE
            pltpu.sync_copy(z_hbm.at[pl.ds(base, INIT_SLICE)],
                            acc.at[pl.ds(base, INIT_SLICE)])

            plsc.subcore_barrier()

            lane = lax.iota(jnp.int32, LANES)

            def row_body(r, _):
                for q in range(128 // LANES):
                    off = r * 128 + q * LANES
                    sv = src_v[pl.ds(off, LANES)]
                    dv = dst_v[pl.ds(off, LANES)]
                    rel = dv - lo
                    inr = (rel >= 0) & (rel < CHUNK_ROWS)
                    trash = CHUNK + s * 128 + q * LANES + lane
                    flat = jnp.where(inr, rel * N_NODES + sv, trash)
                    flat_v[r, pl.ds(q * LANES, LANES)] = flat
                return 0

            lax.fori_loop(0, IDX_ROWS, row_body, 0)

            def sc_body(j, _):
                descs = [
                    pltpu.async_copy(ones_v.at[t],
                                     acc.at[flat_v.at[j * 8 + t]], sem,
                                     add=True)
                    for t in range(8)
                ]
                for dsc in descs:
                    dsc.wait()
                return 0

            lax.fori_loop(0, IDX_ROWS // 8, sc_body, 0)
            for j in range(IDX_ROWS - 4, IDX_ROWS):
                pltpu.sync_copy(acc.at[flat_v.at[j]], drain_v)
            plsc.subcore_barrier()

            pl.delay(2000)
            obase = s * OUT_SLICE
            for t in range(4):
                pltpu.sync_copy(
                    acc.at[pl.ds(obase + t * 16384, 16384)],
                    a_hbm.at[pl.ds(chunk_id * CHUNK + obase + t * 16384,
                                   16384)])
            pltpu.sync_copy(
                acc.at[pl.ds(obase + 4 * 16384, 8448)],
                a_hbm.at[pl.ds(chunk_id * CHUNK + obase + 4 * 16384, 8448)])

            plsc.subcore_barrier()

    return counts_kernel(edge_src, edge_dst, zeros)


def _policy_sample(A, loc, W_embed, b_embed2d, W_self, W_nei, b_gnn2d, W_bi,
                   gum, ag_order, continuing, prev, interpret=False):
    nblk = N_NODES // 128

    def body(a_ref, loc_ref, we_ref, be_ref, ws_ref, wn_ref, bg_ref, wb_ref,
             gum_ref, ago_ref, cont_ref, prev_ref, out_ref, h_s, h2_s,
             lut_ref):
        i = pl.program_id(0)

        @pl.when(i == 0)
        def _embed_step():
            x0 = loc_ref[:, 0:1].astype(jnp.bfloat16).astype(jnp.float32)
            x1 = loc_ref[:, 1:2].astype(jnp.bfloat16).astype(jnp.float32)
            w0 = we_ref[0:1, :].astype(jnp.bfloat16).astype(jnp.float32)
            w1 = we_ref[1:2, :].astype(jnp.bfloat16).astype(jnp.float32)
            h_s[...] = (x0 * w0 + x1 * w1) + be_ref[...]

        @pl.when(i < nblk)
        def _gnn_step():
            a_blk = a_ref[...]
            deg = jnp.sum(a_blk, axis=1, keepdims=True)
            agg = jax.lax.dot_general(
                a_blk, h_s[...], (((1,), (0,)), ((), ())),
                precision=jax.lax.Precision.HIGHEST,
                preferred_element_type=jnp.float32)
            norm = agg / jnp.maximum(deg, 1.0)
            h_blk = h_s[pl.ds(i * 128, 128), :]
            pre = (jax.lax.dot_general(
                h_blk, ws_ref[...], (((1,), (0,)), ((), ())),
                preferred_element_type=jnp.float32) +
                   jax.lax.dot_general(
                norm, wn_ref[...], (((1,), (0,)), ((), ())),
                preferred_element_type=jnp.float32) + bg_ref[...])
            h2_s[pl.ds(i * 128, 128), :] = jnp.maximum(pre, 0.0) + h_blk

        @pl.when(i == nblk)
        def _sample_step():
            ag = h2_s[0:N_AG, :]
            tasks = h2_s[N_AG:N_NODES, :]
            t = jax.lax.dot_general(
                ag, wb_ref[...], (((1,), (0,)), ((), ())),
                preferred_element_type=jnp.float32)
            logits = jax.lax.dot_general(
                t, tasks, (((1,), (1,)), ((), ())),
                preferred_element_type=jnp.float32)
            m = jnp.max(logits, axis=1, keepdims=True)
            e = jnp.exp(logits - m)
            pol = e / jnp.sum(e, axis=1, keepdims=True)
            colid = lax.broadcasted_iota(jnp.int32, (N_AG, N_TASK), 1)
            pol = jnp.where(colid == N_TASK - 1, jnp.float32(1e-5), pol)
            lut_ref[...] = jnp.log(pol + 1e-12)
            log_masked = jnp.log(jnp.float32(1e-12))

            col1 = lax.broadcasted_iota(jnp.int32, (1, N_TASK), 1)
            outid = lax.broadcasted_iota(jnp.int32, (1, N_AG), 1)

            def step(itr, carry):
                mask, acts = carry
                a = ago_ref[itr]
                base = lut_ref[pl.ds(a, 1), :]
                g = gum_ref[pl.ds(itr, 1), :]
                scores = jnp.where(mask != 0, log_masked, base) + g
                mx = jnp.max(scores)
                action = jnp.min(
                    jnp.where(scores == mx, col1, N_TASK)).astype(jnp.int32)
                action = jnp.where(cont_ref[a] != 0, prev_ref[a], action)
                mask = mask | ((col1 == action) &
                               (action != N_TASK - 1)).astype(jnp.int32)
                acts = jnp.where(outid == itr, action, acts)
                return mask, acts

            mask0 = jnp.zeros((1, N_TASK), jnp.int32)
            acts0 = jnp.zeros((1, N_AG), jnp.int32)
            _, acts = lax.fori_loop(0, N_AG, step, (mask0, acts0))
            out_ref[...] = acts

    return pl.pallas_call(
        body,
        grid=(nblk + 1,),
        in_specs=[
            pl.BlockSpec((128, N_NODES), lambda i: (jnp.minimum(i, 16), 0)),
            pl.BlockSpec((N_NODES, 2), lambda i: (0, 0)),
            pl.BlockSpec((2, D), lambda i: (0, 0)),
            pl.BlockSpec((1, D), lambda i: (0, 0)),
            pl.BlockSpec((D, D), lambda i: (0, 0)),
            pl.BlockSpec((D, D), lambda i: (0, 0)),
            pl.BlockSpec((1, D), lambda i: (0, 0)),
            pl.BlockSpec((D, D), lambda i: (0, 0)),
            pl.BlockSpec((N_AG, N_TASK), lambda i: (0, 0)),
            pl.BlockSpec(memory_space=pltpu.SMEM),
            pl.BlockSpec(memory_space=pltpu.SMEM),
            pl.BlockSpec(memory_space=pltpu.SMEM),
        ],
        out_specs=pl.BlockSpec((1, N_AG), lambda i: (0, 0)),
        out_shape=jax.ShapeDtypeStruct((1, N_AG), jnp.int32),
        scratch_shapes=[
            pltpu.VMEM((N_NODES, D), jnp.float32),
            pltpu.VMEM((N_NODES, D), jnp.float32),
            pltpu.VMEM((N_AG, N_TASK), jnp.float32),
        ],
        interpret=interpret,
    )(A, loc, W_embed, b_embed2d, W_self, W_nei, b_gnn2d, W_bi, gum,
      ag_order, continuing, prev)


def kernel(loc, W_embed, b_embed, W_self, W_nei, b_gnn, W_bi, edge_index,
           ag_order, continuing_ag, joint_action_prev):
    edge_src = edge_index[0].reshape(SC_SUBCORES, EDGES_PER_TILE)
    edge_dst = edge_index[1].reshape(SC_SUBCORES, EDGES_PER_TILE)
    zeros = jnp.zeros((ACC_LEN,), jnp.float32)
    a_flat = _build_counts(edge_src.astype(jnp.int32),
                           edge_dst.astype(jnp.int32), zeros)
    A = a_flat.reshape(N_NODES, N_NODES)
    acts = _policy_sample(A, loc, W_embed, b_embed.reshape(1, D), W_self,
                          W_nei, b_gnn.reshape(1, D), W_bi,
                          jnp.asarray(_GUMBEL), ag_order.astype(jnp.int32),
                          continuing_ag.astype(jnp.int32),
                          joint_action_prev.astype(jnp.int32))
    return acts.reshape(N_AG)


# speedup vs baseline: 3.2807x; 1.0229x over previous
import functools

import jax
import jax.numpy as jnp
import numpy as np
from jax import lax
from jax.experimental import pallas as pl
from jax.experimental.pallas import tpu as pltpu
from jax.experimental.pallas import tpu_sc as plsc

N_AG = 128
N_TASK = 2048
N_NODES = N_AG + N_TASK
N_EDGES = 262144
D = 512

SC_CORES = 2
SC_SUBCORES = 16
LANES = 16

EDGES_PER_TILE = N_EDGES // SC_SUBCORES
IDX_ROWS = EDGES_PER_TILE // 128
CHUNK_ROWS = N_NODES // 4
CHUNK = CHUNK_ROWS * N_NODES


def _threefry2x32(k1, k2, x0, x1):
    rot = ((13, 15, 26, 6), (17, 29, 16, 24))
    ks = (np.uint32(k1), np.uint32(k2),
          np.uint32(np.uint32(k1) ^ np.uint32(k2) ^ np.uint32(0x1BD11BDA)))
    x0 = (x0 + ks[0]).astype(np.uint32)
    x1 = (x1 + ks[1]).astype(np.uint32)
    for g in range(5):
        for r in rot[g % 2]:
            x0 = (x0 + x1).astype(np.uint32)
            x1 = ((x1 << np.uint32(r)) | (x1 >> np.uint32(32 - r))).astype(
                np.uint32)
            x1 = (x0 ^ x1).astype(np.uint32)
        x0 = (x0 + ks[(g + 1) % 3]).astype(np.uint32)
        x1 = (x1 + ks[(g + 2) % 3] + np.uint32(g + 1)).astype(np.uint32)
    return x0, x1


def _make_gumbel_table() -> np.ndarray:
    tiny = np.float32(np.finfo(np.float32).tiny)
    k1, k2 = np.uint32(0), np.uint32(42)
    rows = []
    for _ in range(N_AG):
        b1, b2 = _threefry2x32(k1, k2, np.zeros(2, np.uint32),
                               np.arange(2, dtype=np.uint32))
        k1, k2 = b1[0], b2[0]
        s1, s2 = b1[1], b2[1]
        r1, r2 = _threefry2x32(s1, s2, np.zeros(N_TASK, np.uint32),
                               np.arange(N_TASK, dtype=np.uint32))
        bits = (r1 ^ r2).astype(np.uint32)
        fb = ((bits >> np.uint32(9)) | np.uint32(0x3F800000)).astype(np.uint32)
        u = fb.view(np.float32) - np.float32(1.0)
        u = np.maximum(tiny, (u * (np.float32(1.0) - tiny) + tiny))
        g = -np.log(-np.log(u.astype(np.float64)))
        rows.append(g.astype(np.float32))
    return np.stack(rows)


_GUMBEL = _make_gumbel_table()


ACC_LEN = CHUNK + SC_SUBCORES * 128
INIT_SLICE = ACC_LEN // SC_SUBCORES
OUT_SLICE = CHUNK // SC_SUBCORES


def _build_counts(edge_src, edge_dst, zeros):
    mesh = plsc.VectorSubcoreMesh(core_axis_name="c", subcore_axis_name="s")

    @functools.partial(
        pl.kernel,
        mesh=mesh,
        out_type=jax.ShapeDtypeStruct((N_NODES * N_NODES,), jnp.float32),
        scratch_types=[
            pltpu.VMEM((EDGES_PER_TILE,), jnp.int32),
            pltpu.VMEM((EDGES_PER_TILE,), jnp.int32),
            pltpu.VMEM((IDX_ROWS, 128), jnp.int32),
            pltpu.VMEM((8, 128), jnp.float32),
            pltpu.VMEM((128,), jnp.float32),
            pltpu.VMEM_SHARED((ACC_LEN,), jnp.float32),
            pltpu.SemaphoreType.DMA,
        ],
    )
    def counts_kernel(src_hbm, dst_hbm, z_hbm, a_hbm, src_v, dst_v, flat_v,
                      ones_v, drain_v, acc, sem):
        c = lax.axis_index("c")
        s = lax.axis_index("s")
        pltpu.sync_copy(src_hbm.at[s], src_v)
        pltpu.sync_copy(dst_hbm.at[s], dst_v)

        for r in range(8):
            for q in range(128 // LANES):
                ones_v[r, pl.ds(q * LANES, LANES)] = jnp.full(
                    (LANES,), 1.0, jnp.float32)

        for k in range(2):
            chunk_id = c * 2 + k
            lo = chunk_id * CHUNK_ROWS

            base = s * INIT_SLICE
            pltpu.sync_copy(z_hbm.at[pl.ds(base, INIT_SLICE)],
                            acc.at[pl.ds(base, INIT_SLICE)])

            plsc.subcore_barrier()

            lane = lax.iota(jnp.int32, LANES)

            def row_body(r, _):
                for q in range(128 // LANES):
                    off = r * 128 + q * LANES
                    sv = src_v[pl.ds(off, LANES)]
                    dv = dst_v[pl.ds(off, LANES)]
                    rel = dv - lo
                    inr = (rel >= 0) & (rel < CHUNK_ROWS)
                    trash = CHUNK + s * 128 + q * LANES + lane
                    flat = jnp.where(inr, rel * N_NODES + sv, trash)
                    flat_v[r, pl.ds(q * LANES, LANES)] = flat
                return 0

            lax.fori_loop(0, IDX_ROWS, row_body, 0)

            def sc_body(j, _):
                descs = [
                    pltpu.async_copy(ones_v.at[t],
                                     acc.at[flat_v.at[j * 8 + t]], sem,
                                     add=True)
                    for t in range(8)
                ]
                for dsc in descs:
                    dsc.wait()
                return 0

            lax.fori_loop(0, IDX_ROWS // 8, sc_body, 0)
            for j in range(IDX_ROWS - 4, IDX_ROWS):
                pltpu.sync_copy(acc.at[flat_v.at[j]], drain_v)
            plsc.subcore_barrier()

            pl.delay(2000)
            obase = s * OUT_SLICE
            for t in range(4):
                pltpu.sync_copy(
                    acc.at[pl.ds(obase + t * 16384, 16384)],
                    a_hbm.at[pl.ds(chunk_id * CHUNK + obase + t * 16384,
                                   16384)])
            pltpu.sync_copy(
                acc.at[pl.ds(obase + 4 * 16384, 8448)],
                a_hbm.at[pl.ds(chunk_id * CHUNK + obase + 4 * 16384, 8448)])

            plsc.subcore_barrier()

    return counts_kernel(edge_src, edge_dst, zeros)


def _policy_sample(A, loc, W_embed, b_embed2d, W_self, W_nei, b_gnn2d, W_bi,
                   gum, ag_order, continuing, prev, interpret=False):
    nblk = N_NODES // 128

    def body(a_ref, loc_ref, we_ref, be_ref, ws_ref, wn_ref, bg_ref, wb_ref,
             gum_ref, ago_ref, cont_ref, prev_ref, out_ref, h_s, h2_s,
             lut_ref):
        i = pl.program_id(0)

        @pl.when(i == 0)
        def _embed_step():
            x0 = loc_ref[:, 0:1].astype(jnp.bfloat16).astype(jnp.float32)
            x1 = loc_ref[:, 1:2].astype(jnp.bfloat16).astype(jnp.float32)
            w0 = we_ref[0:1, :].astype(jnp.bfloat16).astype(jnp.float32)
            w1 = we_ref[1:2, :].astype(jnp.bfloat16).astype(jnp.float32)
            h_s[...] = (x0 * w0 + x1 * w1) + be_ref[...]

        @pl.when(i < nblk)
        def _gnn_step():
            a_blk = a_ref[...]
            deg = jnp.sum(a_blk, axis=1, keepdims=True)
            agg = jax.lax.dot_general(
                a_blk, h_s[...], (((1,), (0,)), ((), ())),
                precision=jax.lax.Precision.HIGHEST,
                preferred_element_type=jnp.float32)
            norm = agg / jnp.maximum(deg, 1.0)
            h_blk = h_s[pl.ds(i * 128, 128), :]
            pre = (jax.lax.dot_general(
                h_blk, ws_ref[...], (((1,), (0,)), ((), ())),
                preferred_element_type=jnp.float32) +
                   jax.lax.dot_general(
                norm, wn_ref[...], (((1,), (0,)), ((), ())),
                preferred_element_type=jnp.float32) + bg_ref[...])
            h2_s[pl.ds(i * 128, 128), :] = jnp.maximum(pre, 0.0) + h_blk

        @pl.when(i == nblk)
        def _sample_step():
            ag = h2_s[0:N_AG, :]
            tasks = h2_s[N_AG:N_NODES, :]
            t = jax.lax.dot_general(
                ag, wb_ref[...], (((1,), (0,)), ((), ())),
                preferred_element_type=jnp.float32)
            logits = jax.lax.dot_general(
                t, tasks, (((1,), (1,)), ((), ())),
                preferred_element_type=jnp.float32)
            m = jnp.max(logits, axis=1, keepdims=True)
            e = jnp.exp(logits - m)
            pol = e / jnp.sum(e, axis=1, keepdims=True)
            colid = lax.broadcasted_iota(jnp.int32, (N_AG, N_TASK), 1)
            pol = jnp.where(colid == N_TASK - 1, jnp.float32(1e-5), pol)
            lut_ref[...] = jnp.log(pol + 1e-12).reshape(N_AG * 8,
                                                        N_TASK // 8)
            log_masked = jnp.log(jnp.float32(1e-12))

            col2 = (lax.broadcasted_iota(jnp.int32, (8, N_TASK // 8), 0) *
                    (N_TASK // 8) +
                    lax.broadcasted_iota(jnp.int32, (8, N_TASK // 8), 1))
            outid = lax.broadcasted_iota(jnp.int32, (1, N_AG), 1)

            def step(itr, carry):
                mask, acts = carry
                a = ago_ref[itr]
                base = lut_ref[pl.ds(a * 8, 8), :]
                g = gum_ref[pl.ds(itr * 8, 8), :]
                scores = jnp.where(mask != 0, log_masked, base) + g
                mx = jnp.max(scores)
                action = jnp.min(
                    jnp.where(scores == mx, col2, N_TASK)).astype(jnp.int32)
                action = jnp.where(cont_ref[a] != 0, prev_ref[a], action)
                mask = mask | ((col2 == action) &
                               (action != N_TASK - 1)).astype(jnp.int32)
                acts = jnp.where(outid == itr, action, acts)
                return mask, acts

            mask0 = jnp.zeros((8, N_TASK // 8), jnp.int32)
            acts0 = jnp.zeros((1, N_AG), jnp.int32)
            _, acts = lax.fori_loop(0, N_AG, step, (mask0, acts0))
            out_ref[...] = acts

    return pl.pallas_call(
        body,
        grid=(nblk + 1,),
        in_specs=[
            pl.BlockSpec((128, N_NODES), lambda i: (jnp.minimum(i, 16), 0)),
            pl.BlockSpec((N_NODES, 2), lambda i: (0, 0)),
            pl.BlockSpec((2, D), lambda i: (0, 0)),
            pl.BlockSpec((1, D), lambda i: (0, 0)),
            pl.BlockSpec((D, D), lambda i: (0, 0)),
            pl.BlockSpec((D, D), lambda i: (0, 0)),
            pl.BlockSpec((1, D), lambda i: (0, 0)),
            pl.BlockSpec((D, D), lambda i: (0, 0)),
            pl.BlockSpec((N_AG * 8, N_TASK // 8), lambda i: (0, 0)),
            pl.BlockSpec(memory_space=pltpu.SMEM),
            pl.BlockSpec(memory_space=pltpu.SMEM),
            pl.BlockSpec(memory_space=pltpu.SMEM),
        ],
        out_specs=pl.BlockSpec((1, N_AG), lambda i: (0, 0)),
        out_shape=jax.ShapeDtypeStruct((1, N_AG), jnp.int32),
        scratch_shapes=[
            pltpu.VMEM((N_NODES, D), jnp.float32),
            pltpu.VMEM((N_NODES, D), jnp.float32),
            pltpu.VMEM((N_AG * 8, N_TASK // 8), jnp.float32),
        ],
        interpret=interpret,
    )(A, loc, W_embed, b_embed2d, W_self, W_nei, b_gnn2d, W_bi, gum,
      ag_order, continuing, prev)


def kernel(loc, W_embed, b_embed, W_self, W_nei, b_gnn, W_bi, edge_index,
           ag_order, continuing_ag, joint_action_prev):
    edge_src = edge_index[0].reshape(SC_SUBCORES, EDGES_PER_TILE)
    edge_dst = edge_index[1].reshape(SC_SUBCORES, EDGES_PER_TILE)
    zeros = jnp.zeros((ACC_LEN,), jnp.float32)
    a_flat = _build_counts(edge_src.astype(jnp.int32),
                           edge_dst.astype(jnp.int32), zeros)
    A = a_flat.reshape(N_NODES, N_NODES)
    acts = _policy_sample(A, loc, W_embed, b_embed.reshape(1, D), W_self,
                          W_nei, b_gnn.reshape(1, D), W_bi,
                          jnp.asarray(_GUMBEL).reshape(N_AG * 8, N_TASK // 8),
                          ag_order.astype(jnp.int32),
                          continuing_ag.astype(jnp.int32),
                          joint_action_prev.astype(jnp.int32))
    return acts.reshape(N_AG)
